# baseline (speedup 1.0000x reference)
import functools

import jax
import jax.numpy as jnp
from jax.experimental import pallas as pl
from jax.experimental.pallas import tpu as pltpu

_NCORES = 2


def _colsum_body(x_ref, o_ref):
    j = pl.program_id(1)
    s = jnp.sum(x_ref[...].reshape(-1, 8, x_ref.shape[1]), axis=0)

    @pl.when(j == 0)
    def _():
        o_ref[...] = s

    @pl.when(j > 0)
    def _():
        o_ref[...] += s


def _fused_body(x_ref, w1_ref, w2_ref, bias_ref, part_ref, o_ref, *, inv_n):
    pooled = jnp.sum(part_ref[...], axis=0, keepdims=True)
    transmit = jnp.dot(pooled, w2_ref[...],
                       preferred_element_type=jnp.float32) * inv_n
    o_ref[...] = (jnp.dot(x_ref[...], w1_ref[...],
                          preferred_element_type=jnp.float32)
                  + transmit + bias_ref[...])


@jax.jit
def kernel(x, w1, w2, bias):
    n, d_in = x.shape
    d_out = w1.shape[1]
    rows_per_core = n // _NCORES
    block = None
    for cand in (8000, 5000, 4000, 2000, 1000, 500, 200, 100, 50, 25, 10, 8):
        if rows_per_core % cand == 0 and cand % 8 == 0:
            block = cand
            break
    if block is None:
        block = rows_per_core
    nb = rows_per_core // block

    partials = pl.pallas_call(
        _colsum_body,
        grid=(_NCORES, nb),
        in_specs=[pl.BlockSpec((block, d_in), lambda i, j: (i * nb + j, 0))],
        out_specs=pl.BlockSpec((8, d_in), lambda i, j: (i, 0)),
        out_shape=jax.ShapeDtypeStruct((_NCORES * 8, d_in), jnp.float32),
        compiler_params=pltpu.CompilerParams(
            dimension_semantics=("parallel", "arbitrary")),
    )(x)

    out = pl.pallas_call(
        functools.partial(_fused_body, inv_n=1.0 / n),
        grid=(_NCORES, nb),
        in_specs=[
            pl.BlockSpec((block, d_in), lambda i, j: (i * nb + j, 0)),
            pl.BlockSpec((d_in, d_out), lambda i, j: (0, 0)),
            pl.BlockSpec((d_in, d_out), lambda i, j: (0, 0)),
            pl.BlockSpec((1, d_out), lambda i, j: (0, 0)),
            pl.BlockSpec((_NCORES * 8, d_in), lambda i, j: (0, 0)),
        ],
        out_specs=pl.BlockSpec((block, d_out), lambda i, j: (i * nb + j, 0)),
        out_shape=jax.ShapeDtypeStruct((n, d_out), jnp.float32),
        compiler_params=pltpu.CompilerParams(
            dimension_semantics=("parallel", "arbitrary")),
    )(x, w1, w2, bias, partials)
    return out


# speedup vs baseline: 1.2956x; 1.2956x over previous
import functools

import jax
import jax.numpy as jnp
from jax.experimental import pallas as pl
from jax.experimental.pallas import tpu as pltpu

_NCORES = 2
_NSTREAM = 4


def _colsum_body(*refs):
    x_refs, o_ref = refs[:-1], refs[-1]
    j = pl.program_id(1)
    s = x_refs[0][...].reshape(-1, 8, x_refs[0].shape[1]).sum(axis=0)
    for r in x_refs[1:]:
        s = s + r[...].reshape(-1, 8, r.shape[1]).sum(axis=0)

    @pl.when(j == 0)
    def _():
        o_ref[...] = s

    @pl.when(j > 0)
    def _():
        o_ref[...] += s


def _fused_body(*refs, inv_n, nstream, block):
    x_refs = refs[:nstream]
    w1_ref, w2_ref, bias_ref, part_ref = refs[nstream:nstream + 4]
    o_ref = refs[-1]
    pooled = jnp.sum(part_ref[...], axis=0, keepdims=True)
    transmit = (jnp.dot(pooled, w2_ref[...],
                        preferred_element_type=jnp.float32) * inv_n
                + bias_ref[...])
    w1 = w1_ref[...]
    for q, x_ref in enumerate(x_refs):
        o_ref[q * block:(q + 1) * block, :] = (
            jnp.dot(x_ref[...], w1, preferred_element_type=jnp.float32)
            + transmit)


@jax.jit
def kernel(x, w1, w2, bias):
    n, d_in = x.shape
    d_out = w1.shape[1]
    rows_per_core = n // _NCORES
    block = None
    for cand in (8000, 5000, 4000, 2000, 1000, 500, 200, 100, 50, 25, 10, 8):
        if rows_per_core % cand == 0 and cand % 8 == 0:
            block = cand
            break
    if block is None:
        block = rows_per_core
    nb = rows_per_core // block
    nstream = _NSTREAM
    while nb % nstream != 0:
        nstream //= 2
    nbs = nb // nstream

    def _xmap(q):
        return lambda i, j: (i * nb + q * nbs + j, 0)

    partials = pl.pallas_call(
        _colsum_body,
        grid=(_NCORES, nbs),
        in_specs=[pl.BlockSpec((block, d_in), _xmap(q)) for q in range(nstream)],
        out_specs=pl.BlockSpec((8, d_in), lambda i, j: (i, 0)),
        out_shape=jax.ShapeDtypeStruct((_NCORES * 8, d_in), jnp.float32),
        compiler_params=pltpu.CompilerParams(
            dimension_semantics=("parallel", "arbitrary")),
    )(*([x] * nstream))

    out = pl.pallas_call(
        functools.partial(_fused_body, inv_n=1.0 / n, nstream=nstream,
                          block=block),
        grid=(_NCORES, nbs),
        in_specs=(
            [pl.BlockSpec((block, d_in),
                          lambda i, j, q=q: (i * nb + j * nstream + q, 0))
             for q in range(nstream)]
            + [
                pl.BlockSpec((d_in, d_out), lambda i, j: (0, 0)),
                pl.BlockSpec((d_in, d_out), lambda i, j: (0, 0)),
                pl.BlockSpec((1, d_out), lambda i, j: (0, 0)),
                pl.BlockSpec((_NCORES * 8, d_in), lambda i, j: (0, 0)),
            ]
        ),
        out_specs=pl.BlockSpec((block * nstream, d_out),
                               lambda i, j: (i * nbs + j, 0)),
        out_shape=jax.ShapeDtypeStruct((n, d_out), jnp.float32),
        compiler_params=pltpu.CompilerParams(
            dimension_semantics=("parallel", "arbitrary")),
    )(*([x] * nstream), w1, w2, bias, partials)
    return out


# speedup vs baseline: 1.6469x; 1.2712x over previous
import functools

import jax
import jax.numpy as jnp
from jax.experimental import pallas as pl
from jax.experimental.pallas import tpu as pltpu

_R = 6
_W = 6


def _body(x_hbm, w1_ref, w2_ref, bias_ref, o_hbm, ring, cache, o_ring,
          in_sem, out_sem, *, n, block, nb):
    d = x_hbm.shape[1]
    inv_n = 1.0 / n

    def start_in(g):
        pltpu.make_async_copy(
            x_hbm.at[pl.ds(g * block, block)],
            ring.at[jax.lax.rem(g, _R)],
            in_sem.at[jax.lax.rem(g, _R)],
        ).start()

    def wait_in(g):
        pltpu.make_async_copy(
            x_hbm.at[pl.ds(0, block)],
            ring.at[jax.lax.rem(g, _R)],
            in_sem.at[jax.lax.rem(g, _R)],
        ).wait()

    def start_out(k):
        pltpu.make_async_copy(
            o_ring.at[jax.lax.rem(k, _W)],
            o_hbm.at[pl.ds(k * block, block)],
            out_sem.at[jax.lax.rem(k, _W)],
        ).start()

    def wait_out(k):
        pltpu.make_async_copy(
            o_ring.at[jax.lax.rem(k, _W)],
            o_hbm.at[pl.ds(0, block)],
            out_sem.at[jax.lax.rem(k, _W)],
        ).wait()

    for g in range(min(_R, nb)):
        start_in(g)

    def p1(k, acc):
        wait_in(k)
        blk = ring[jax.lax.rem(k, _R)]
        acc = acc + jnp.sum(blk.reshape(-1, 8, d), axis=0)
        cache[k] = blk.astype(jnp.bfloat16)

        @pl.when(k + _R < nb)
        def _():
            start_in(k + _R)

        return acc

    acc = jax.lax.fori_loop(0, nb, p1, jnp.zeros((8, d), jnp.float32))

    pooled = jnp.sum(acc, axis=0, keepdims=True)
    transmit = (jnp.dot(pooled, w2_ref[...],
                        preferred_element_type=jnp.float32) * inv_n
                + bias_ref[...])
    w1b = w1_ref[...].astype(jnp.bfloat16)

    def p2(k, _):
        @pl.when(k >= _W)
        def _():
            wait_out(k)

        o_ring[jax.lax.rem(k, _W)] = (
            jnp.dot(cache[k], w1b, preferred_element_type=jnp.float32)
            + transmit)
        start_out(k)
        return 0

    jax.lax.fori_loop(0, nb, p2, 0)
    for s in range(min(_W, nb)):
        wait_out(s)


@jax.jit
def kernel(x, w1, w2, bias):
    n, d_in = x.shape
    d_out = w1.shape[1]
    block = None
    for cand in (2000, 1600, 1000, 800, 400, 200, 80, 16):
        if n % cand == 0:
            block = cand
            break
    if block is None:
        block = n
    nb = n // block

    out = pl.pallas_call(
        functools.partial(_body, n=n, block=block, nb=nb),
        in_specs=[
            pl.BlockSpec(memory_space=pl.ANY),
            pl.BlockSpec((d_in, d_out), lambda: (0, 0)),
            pl.BlockSpec((d_in, d_out), lambda: (0, 0)),
            pl.BlockSpec((1, d_out), lambda: (0, 0)),
        ],
        out_specs=pl.BlockSpec(memory_space=pl.ANY),
        out_shape=jax.ShapeDtypeStruct((n, d_out), jnp.float32),
        scratch_shapes=[
            pltpu.VMEM((_R, block, d_in), jnp.float32),
            pltpu.VMEM((nb, block, d_in), jnp.bfloat16),
            pltpu.VMEM((_W, block, d_out), jnp.float32),
            pltpu.SemaphoreType.DMA((_R,)),
            pltpu.SemaphoreType.DMA((_W,)),
        ],
        compiler_params=pltpu.CompilerParams(
            vmem_limit_bytes=128 * 1024 * 1024),
    )(x, w1, w2, bias)
    return out
